# Initial kernel scaffold; baseline (speedup 1.0000x reference)
import jax
import jax.numpy as jnp
from jax.experimental import pallas as pl


def kernel(hidden_states, Wg, W_up, W_gate, W_down):
    raise NotImplementedError("write your pallas kernel here")


# speedup vs baseline: 1.4942x; 1.4942x over previous
import functools

import jax
import jax.numpy as jnp
from jax.experimental import pallas as pl
from jax.experimental.pallas import tpu as pltpu

B, S, D = 1, 2048, 1024
FF = 3584
E = 8
TOP_K = 2

FFC = 512
NF = FF // FFC


def _router_body(h_ref, wg_ref, ew_ref):
    h = h_ref[...]
    wg = wg_ref[...]
    logits = jnp.dot(h, wg, preferred_element_type=jnp.float32)
    m = jnp.max(logits, axis=1, keepdims=True)
    ex = jnp.exp(logits - m)
    p = ex / jnp.sum(ex, axis=1, keepdims=True)
    idx = jax.lax.broadcasted_iota(jnp.int32, (S, E), 1)
    v0 = jnp.max(p, axis=1, keepdims=True)
    e0 = jnp.min(jnp.where(p == v0, idx, E), axis=1, keepdims=True)
    p1 = jnp.where(idx == e0, -jnp.inf, p)
    v1 = jnp.max(p1, axis=1, keepdims=True)
    e1 = jnp.min(jnp.where(p1 == v1, idx, E), axis=1, keepdims=True)
    s = v0 + v1
    ew_ref[...] = jnp.where(idx == e0, v0 / s, 0.0) + jnp.where(idx == e1, v1 / s, 0.0)


def _moe_body(ew_ref, h_ref, wup_ref, wgate_ref, wdown_ref, out_ref):
    e = pl.program_id(0)
    f = pl.program_id(1)

    @pl.when((e == 0) & (f == 0))
    def _():
        out_ref[...] = jnp.zeros_like(out_ref)

    x = h_ref[...]
    up = jnp.dot(x, wup_ref[0], preferred_element_type=jnp.float32)
    gate = jnp.dot(x, wgate_ref[0], preferred_element_type=jnp.float32)
    z = up * jax.nn.sigmoid(up) * gate

    idx = jax.lax.broadcasted_iota(jnp.int32, (S, E), 1)
    w_col = jnp.sum(jnp.where(idx == e, ew_ref[...], 0.0), axis=1, keepdims=True)
    out_ref[...] += jnp.dot(z * w_col, wdown_ref[0], preferred_element_type=jnp.float32)


@jax.jit
def _run(h2d, Wg, W_up, W_gate, W_down):
    ew = pl.pallas_call(
        _router_body,
        out_shape=jax.ShapeDtypeStruct((S, E), jnp.float32),
    )(h2d, Wg)

    out = pl.pallas_call(
        _moe_body,
        grid=(E, NF),
        in_specs=[
            pl.BlockSpec((S, E), lambda e, f: (0, 0)),
            pl.BlockSpec((S, D), lambda e, f: (0, 0)),
            pl.BlockSpec((1, D, FFC), lambda e, f: (e, 0, f)),
            pl.BlockSpec((1, D, FFC), lambda e, f: (e, 0, f)),
            pl.BlockSpec((1, FFC, D), lambda e, f: (e, f, 0)),
        ],
        out_specs=pl.BlockSpec((S, D), lambda e, f: (0, 0)),
        out_shape=jax.ShapeDtypeStruct((S, D), jnp.float32),
    )(ew, h2d, W_up, W_gate, W_down)
    return out


def kernel(hidden_states, Wg, W_up, W_gate, W_down):
    h2d = hidden_states.reshape(-1, D)
    out = _run(h2d, Wg, W_up, W_gate, W_down)
    return out.reshape(hidden_states.shape)
